# baseline (speedup 1.0000x reference)
import jax
import jax.numpy as jnp
from jax.experimental import pallas as pl
from jax.experimental.pallas import tpu as pltpu

DIM = 64
EPS = 1e-07
BLOCK_ROWS = 4096


def _expmap_body(x_ref, o_ref):
    v = x_ref[...] * 0.1
    n2 = jnp.sum(v * v, axis=-1, keepdims=True)
    n = jnp.maximum(jnp.sqrt(n2), EPS)
    o_ref[...] = v * (jnp.tanh(n) / n)


def kernel(x):
    orig_shape = x.shape
    rows = x.size // DIM
    x2 = x.reshape(rows, DIM)
    grid = (rows // BLOCK_ROWS,)
    out = pl.pallas_call(
        _expmap_body,
        grid=grid,
        in_specs=[pl.BlockSpec((BLOCK_ROWS, DIM), lambda i: (i, 0))],
        out_specs=pl.BlockSpec((BLOCK_ROWS, DIM), lambda i: (i, 0)),
        out_shape=jax.ShapeDtypeStruct((rows, DIM), jnp.float32),
        compiler_params=pltpu.CompilerParams(
            dimension_semantics=("arbitrary",),
        ),
    )(x2)
    return out.reshape(orig_shape)


# speedup vs baseline: 1.5799x; 1.5799x over previous
import jax
import jax.numpy as jnp
from jax.experimental import pallas as pl
from jax.experimental.pallas import tpu as pltpu

DIM = 64
EPS = 1e-07
BLOCK_ROWS = 4096


def _expmap_body(x_ref, o_ref):
    v = x_ref[...] * 0.1
    n2 = jnp.sum(v * v, axis=-1, keepdims=True)
    n = jnp.maximum(jnp.sqrt(n2), EPS)
    o_ref[...] = v * (jnp.tanh(n) / n)


BLOCK_B = 128


def kernel(x):
    b, s, d = x.shape
    grid = (b // BLOCK_B,)
    out = pl.pallas_call(
        _expmap_body,
        grid=grid,
        in_specs=[pl.BlockSpec((BLOCK_B, s, d), lambda i: (i, 0, 0))],
        out_specs=pl.BlockSpec((BLOCK_B, s, d), lambda i: (i, 0, 0)),
        out_shape=jax.ShapeDtypeStruct((b, s, d), jnp.float32),
        compiler_params=pltpu.CompilerParams(
            dimension_semantics=("arbitrary",),
        ),
    )(x)
    return out
